# baseline (speedup 1.0000x reference)
import functools

import jax
import jax.numpy as jnp
from jax import lax
from jax.experimental import pallas as pl
from jax.experimental.pallas import tpu as pltpu
from jax.experimental.pallas import tpu_sc as plsc

IN_F = 256
OUT_F = 256
NUM_E = 64
BATCH = 2048

TILE = 256
NUM_TILES = BATCH // TILE
G = NUM_TILES + NUM_E - 1

SC_CORES = 2
SC_SUBCORES = 16
SC_WORKERS = SC_CORES * SC_SUBCORES


def _sc_row_gather(table, idx):
    n = idx.shape[0]
    d = table.shape[1]
    rows_per_w = n // SC_WORKERS
    mesh = plsc.VectorSubcoreMesh(core_axis_name="c", subcore_axis_name="s")

    @functools.partial(
        pl.kernel,
        out_type=jax.ShapeDtypeStruct((n, d), table.dtype),
        mesh=mesh,
        scratch_types=[
            pltpu.VMEM((rows_per_w,), jnp.int32),
            pltpu.VMEM((rows_per_w, d), table.dtype),
            pltpu.SemaphoreType.DMA,
        ],
    )
    def k(table_hbm, idx_hbm, out_hbm, idx_v, rows_v, sem):
        wid = lax.axis_index("s") * SC_CORES + lax.axis_index("c")
        base = wid * rows_per_w
        pltpu.sync_copy(idx_hbm.at[pl.ds(base, rows_per_w)], idx_v)
        pltpu.async_copy(table_hbm.at[idx_v], rows_v, sem).wait()
        pltpu.sync_copy(rows_v, out_hbm.at[pl.ds(base, rows_per_w)])

    return k(table, idx)


def _gmm_body(tile_r, e_r, lo_r, hi_r, x_ref, w_ref, b_ref, o_ref):
    g = pl.program_id(0)
    t = tile_r[g]
    lo = lo_r[g]
    hi = hi_r[g]
    rows = t * TILE + lax.broadcasted_iota(jnp.int32, (TILE, 1), 0)
    mask = (rows >= lo) & (rows < hi)
    x = jnp.where(mask, x_ref[...], 0.0).astype(jnp.bfloat16)
    acc = lax.dot_general(
        x, w_ref[0], (((1,), (1,)), ((), ())), preferred_element_type=jnp.float32
    )
    acc = acc + jnp.where(mask, b_ref[0], 0.0)
    first = jnp.logical_or(g == 0, t != tile_r[jnp.maximum(g - 1, 0)])

    @pl.when(first)
    def _():
        o_ref[...] = acc

    @pl.when(jnp.logical_not(first))
    def _():
        o_ref[...] += acc


def _grouped_matmul(tile_of, e_of, lo, hi, x_sorted, w3, bias3):
    grid_spec = pltpu.PrefetchScalarGridSpec(
        num_scalar_prefetch=4,
        grid=(G,),
        in_specs=[
            pl.BlockSpec((TILE, IN_F), lambda g, tr, er, lr, hr: (tr[g], 0)),
            pl.BlockSpec((1, OUT_F, IN_F), lambda g, tr, er, lr, hr: (er[g], 0, 0)),
            pl.BlockSpec((1, 1, OUT_F), lambda g, tr, er, lr, hr: (er[g], 0, 0)),
        ],
        out_specs=pl.BlockSpec((TILE, OUT_F), lambda g, tr, er, lr, hr: (tr[g], 0)),
    )
    return pl.pallas_call(
        _gmm_body,
        grid_spec=grid_spec,
        out_shape=jax.ShapeDtypeStruct((BATCH, OUT_F), jnp.float32),
        compiler_params=pltpu.CompilerParams(dimension_semantics=("arbitrary",)),
    )(tile_of, e_of, lo, hi, x_sorted, w3, bias3)


def kernel(input, route_index, weight, weight_fact, bias, bias_fact):
    r = route_index.astype(jnp.int32)
    perm = jnp.argsort(r).astype(jnp.int32)
    inv = jnp.zeros((BATCH,), jnp.int32).at[perm].set(
        jnp.arange(BATCH, dtype=jnp.int32)
    )

    counts = jnp.bincount(r, length=NUM_E)
    off = jnp.concatenate(
        [jnp.zeros((1,), jnp.int32), jnp.cumsum(counts).astype(jnp.int32)]
    )
    first_t = off[:NUM_E] // TILE
    nonempty = counts > 0
    last_t = jnp.where(nonempty, (off[1:] - 1) // TILE, 0)
    ntiles = jnp.where(nonempty, last_t - first_t + 1, 0)
    cum = jnp.cumsum(ntiles)
    starts = cum - ntiles
    gids = jnp.arange(G)
    e_g = jnp.searchsorted(cum, gids, side="right")
    valid = e_g < NUM_E
    e_safe = jnp.where(valid, e_g, 0).astype(jnp.int32)
    t_g = first_t[e_safe] + (gids - starts[e_safe])
    tile_of = jnp.where(valid, t_g, NUM_TILES - 1).astype(jnp.int32)
    lo = jnp.where(valid, off[e_safe], 0).astype(jnp.int32)
    hi = jnp.where(valid, off[e_safe + 1], 0).astype(jnp.int32)

    x_sorted = _sc_row_gather(input, perm)
    w3 = (weight + weight_fact).reshape(NUM_E, OUT_F, IN_F).astype(jnp.bfloat16)
    bias3 = (bias + bias_fact).reshape(NUM_E, 1, OUT_F)
    y_sorted = _grouped_matmul(tile_of, e_safe, lo, hi, x_sorted, w3, bias3)
    return _sc_row_gather(y_sorted, inv)


# speedup vs baseline: 1.0072x; 1.0072x over previous
import functools

import jax
import jax.numpy as jnp
from jax import lax
from jax.experimental import pallas as pl
from jax.experimental.pallas import tpu as pltpu
from jax.experimental.pallas import tpu_sc as plsc

IN_F = 256
OUT_F = 256
NUM_E = 64
BATCH = 2048

TILE = 256
NUM_TILES = BATCH // TILE
G = NUM_TILES + NUM_E - 1

SC_CORES = 2
SC_SUBCORES = 16
SC_WORKERS = SC_CORES * SC_SUBCORES


def _sc_row_gather(table, idx):
    n = idx.shape[0]
    d = table.shape[1]
    rows_per_w = n // SC_WORKERS
    mesh = plsc.VectorSubcoreMesh(core_axis_name="c", subcore_axis_name="s")

    @functools.partial(
        pl.kernel,
        out_type=jax.ShapeDtypeStruct((n, d), table.dtype),
        mesh=mesh,
        scratch_types=[
            pltpu.VMEM((rows_per_w,), jnp.int32),
            pltpu.VMEM((rows_per_w, d), table.dtype),
            pltpu.SemaphoreType.DMA,
        ],
    )
    def k(table_hbm, idx_hbm, out_hbm, idx_v, rows_v, sem):
        wid = lax.axis_index("s") * SC_CORES + lax.axis_index("c")
        base = wid * rows_per_w
        pltpu.sync_copy(idx_hbm.at[pl.ds(base, rows_per_w)], idx_v)
        pltpu.async_copy(table_hbm.at[idx_v], rows_v, sem).wait()
        pltpu.sync_copy(rows_v, out_hbm.at[pl.ds(base, rows_per_w)])

    return k(table, idx)


def _gmm_body(tile_r, e_r, lo_r, hi_r, x_ref, w_ref, b_ref, o_ref):
    g = pl.program_id(0)
    t = tile_r[g]
    lo = lo_r[g]
    hi = hi_r[g]
    base = t * TILE
    rows = base + lax.broadcasted_iota(jnp.int32, (TILE, 1), 0)
    mask = (rows >= lo) & (rows < hi)
    x = jnp.where(mask, x_ref[pl.ds(base, TILE), :], 0.0).astype(jnp.bfloat16)
    acc = lax.dot_general(
        x, w_ref[0], (((1,), (1,)), ((), ())), preferred_element_type=jnp.float32
    )
    brow = b_ref[pl.ds(e_r[g], 1), :]
    acc = acc + jnp.where(mask, brow, 0.0)
    first = jnp.logical_or(g == 0, t != tile_r[jnp.maximum(g - 1, 0)])

    @pl.when(first)
    def _():
        o_ref[pl.ds(base, TILE), :] = acc

    @pl.when(jnp.logical_not(first))
    def _():
        o_ref[pl.ds(base, TILE), :] += acc


def _grouped_matmul(tile_of, e_of, lo, hi, x_sorted, w3, bias2):
    grid_spec = pltpu.PrefetchScalarGridSpec(
        num_scalar_prefetch=4,
        grid=(G,),
        in_specs=[
            pl.BlockSpec((BATCH, IN_F), lambda g, tr, er, lr, hr: (0, 0)),
            pl.BlockSpec((1, OUT_F, IN_F), lambda g, tr, er, lr, hr: (er[g], 0, 0)),
            pl.BlockSpec((NUM_E, OUT_F), lambda g, tr, er, lr, hr: (0, 0)),
        ],
        out_specs=pl.BlockSpec((BATCH, OUT_F), lambda g, tr, er, lr, hr: (0, 0)),
    )
    return pl.pallas_call(
        _gmm_body,
        grid_spec=grid_spec,
        out_shape=jax.ShapeDtypeStruct((BATCH, OUT_F), jnp.float32),
        compiler_params=pltpu.CompilerParams(dimension_semantics=("arbitrary",)),
    )(tile_of, e_of, lo, hi, x_sorted, w3, bias2)


def kernel(input, route_index, weight, weight_fact, bias, bias_fact):
    r = route_index.astype(jnp.int32)
    perm = jnp.argsort(r).astype(jnp.int32)
    inv = jnp.zeros((BATCH,), jnp.int32).at[perm].set(
        jnp.arange(BATCH, dtype=jnp.int32)
    )

    counts = jnp.bincount(r, length=NUM_E)
    off = jnp.concatenate(
        [jnp.zeros((1,), jnp.int32), jnp.cumsum(counts).astype(jnp.int32)]
    )
    first_t = off[:NUM_E] // TILE
    nonempty = counts > 0
    last_t = jnp.where(nonempty, (off[1:] - 1) // TILE, 0)
    ntiles = jnp.where(nonempty, last_t - first_t + 1, 0)
    cum = jnp.cumsum(ntiles)
    starts = cum - ntiles
    gids = jnp.arange(G)
    e_g = jnp.searchsorted(cum, gids, side="right")
    valid = e_g < NUM_E
    e_safe = jnp.where(valid, e_g, 0).astype(jnp.int32)
    t_g = first_t[e_safe] + (gids - starts[e_safe])
    tile_of = jnp.where(valid, t_g, NUM_TILES - 1).astype(jnp.int32)
    lo = jnp.where(valid, off[e_safe], 0).astype(jnp.int32)
    hi = jnp.where(valid, off[e_safe + 1], 0).astype(jnp.int32)

    x_sorted = _sc_row_gather(input, perm)
    w3 = (weight + weight_fact).reshape(NUM_E, OUT_F, IN_F).astype(jnp.bfloat16)
    bias2 = bias + bias_fact
    y_sorted = _grouped_matmul(tile_of, e_safe, lo, hi, x_sorted, w3, bias2)
    return _sc_row_gather(y_sorted, inv)


# speedup vs baseline: 1.2546x; 1.2457x over previous
import functools

import jax
import jax.numpy as jnp
from jax import lax
from jax.experimental import pallas as pl
from jax.experimental.pallas import tpu as pltpu
from jax.experimental.pallas import tpu_sc as plsc

IN_F = 256
OUT_F = 256
NUM_E = 64
BATCH = 2048

TILE = 256
NUM_TILES = BATCH // TILE
G = NUM_TILES + NUM_E - 1

SC_CORES = 2
SC_SUBCORES = 16
SC_WORKERS = SC_CORES * SC_SUBCORES


def _sc_row_gather(table, idx):
    n = idx.shape[0]
    d = table.shape[1]
    rows_per_w = n // SC_WORKERS
    mesh = plsc.VectorSubcoreMesh(core_axis_name="c", subcore_axis_name="s")

    @functools.partial(
        pl.kernel,
        out_type=jax.ShapeDtypeStruct((n, d), table.dtype),
        mesh=mesh,
        scratch_types=[
            pltpu.VMEM((rows_per_w,), jnp.int32),
            pltpu.VMEM((rows_per_w, d), table.dtype),
            pltpu.SemaphoreType.DMA,
        ],
    )
    def k(table_hbm, idx_hbm, out_hbm, idx_v, rows_v, sem):
        wid = lax.axis_index("s") * SC_CORES + lax.axis_index("c")
        base = wid * rows_per_w
        pltpu.sync_copy(idx_hbm.at[pl.ds(base, rows_per_w)], idx_v)
        pltpu.async_copy(table_hbm.at[idx_v], rows_v, sem).wait()
        pltpu.sync_copy(rows_v, out_hbm.at[pl.ds(base, rows_per_w)])

    return k(table, idx)


NBUF = 4


def _gmm_body(
    n_r, tile_r, e_r, lo_r, hi_r, x_ref, w_hbm, b_ref, o_ref, wbuf, sems
):
    n = n_r[0]

    def start(slot, j):
        pltpu.make_async_copy(
            w_hbm.at[pl.ds(e_r[j], 1)], wbuf.at[pl.ds(slot, 1)], sems.at[slot]
        ).start()

    def wait(slot):
        pltpu.make_async_copy(
            w_hbm.at[pl.ds(0, 1)], wbuf.at[pl.ds(slot, 1)], sems.at[slot]
        ).wait()

    o_ref[...] = jnp.zeros((BATCH, OUT_F), jnp.float32)
    for b in range(NBUF):
        @pl.when(b < n)
        def _(b=b):
            start(b, b)

    def round_body(rnd, _):
        j0 = rnd * NBUF
        for b in range(NBUF):
            @pl.when(j0 + b < n)
            def _(b=b):
                j = j0 + b
                wait(b)
                t = tile_r[j]
                lo = lo_r[j]
                hi = hi_r[j]
                base = t * TILE
                rows = base + lax.broadcasted_iota(jnp.int32, (TILE, 1), 0)
                mask = (rows >= lo) & (rows < hi)
                x = jnp.where(mask, x_ref[pl.ds(base, TILE), :], 0.0).astype(
                    jnp.bfloat16
                )
                w = wbuf[pl.ds(b, 1)][0]
                acc = lax.dot_general(
                    x, w, (((1,), (1,)), ((), ())),
                    preferred_element_type=jnp.float32,
                )
                brow = b_ref[pl.ds(e_r[j], 1), :]
                acc = acc + jnp.where(mask, brow, 0.0)
                o_ref[pl.ds(base, TILE), :] += acc

                @pl.when(j + NBUF < n)
                def _():
                    start(b, j + NBUF)
        return 0

    nrounds = (n + NBUF - 1) // NBUF
    lax.fori_loop(0, nrounds, round_body, 0)


def _grouped_matmul(n_items, tile_of, e_of, lo, hi, x_sorted, w3, bias2):
    grid_spec = pltpu.PrefetchScalarGridSpec(
        num_scalar_prefetch=5,
        grid=(1,),
        in_specs=[
            pl.BlockSpec((BATCH, IN_F), lambda g, *_: (0, 0)),
            pl.BlockSpec(memory_space=pltpu.MemorySpace.HBM),
            pl.BlockSpec((NUM_E, OUT_F), lambda g, *_: (0, 0)),
        ],
        out_specs=pl.BlockSpec((BATCH, OUT_F), lambda g, *_: (0, 0)),
        scratch_shapes=[
            pltpu.VMEM((NBUF, OUT_F, IN_F), jnp.bfloat16),
            pltpu.SemaphoreType.DMA((NBUF,)),
        ],
    )
    return pl.pallas_call(
        _gmm_body,
        grid_spec=grid_spec,
        out_shape=jax.ShapeDtypeStruct((BATCH, OUT_F), jnp.float32),
        compiler_params=pltpu.CompilerParams(dimension_semantics=("arbitrary",)),
    )(n_items, tile_of, e_of, lo, hi, x_sorted, w3, bias2)


def kernel(input, route_index, weight, weight_fact, bias, bias_fact):
    r = route_index.astype(jnp.int32)
    perm = jnp.argsort(r).astype(jnp.int32)
    inv = jnp.zeros((BATCH,), jnp.int32).at[perm].set(
        jnp.arange(BATCH, dtype=jnp.int32)
    )

    counts = jnp.bincount(r, length=NUM_E)
    off = jnp.concatenate(
        [jnp.zeros((1,), jnp.int32), jnp.cumsum(counts).astype(jnp.int32)]
    )
    first_t = off[:NUM_E] // TILE
    nonempty = counts > 0
    last_t = jnp.where(nonempty, (off[1:] - 1) // TILE, 0)
    ntiles = jnp.where(nonempty, last_t - first_t + 1, 0)
    cum = jnp.cumsum(ntiles)
    starts = cum - ntiles
    gids = jnp.arange(G)
    e_g = jnp.searchsorted(cum, gids, side="right")
    valid = e_g < NUM_E
    e_safe = jnp.where(valid, e_g, 0).astype(jnp.int32)
    t_g = first_t[e_safe] + (gids - starts[e_safe])
    tile_of = jnp.where(valid, t_g, NUM_TILES - 1).astype(jnp.int32)
    lo = jnp.where(valid, off[e_safe], 0).astype(jnp.int32)
    hi = jnp.where(valid, off[e_safe + 1], 0).astype(jnp.int32)

    x_sorted = _sc_row_gather(input, perm)
    w3 = (weight + weight_fact).reshape(NUM_E, OUT_F, IN_F).astype(jnp.bfloat16)
    bias2 = bias + bias_fact
    n_items = cum[NUM_E - 1 :].astype(jnp.int32)
    y_sorted = _grouped_matmul(
        n_items, tile_of, e_safe, lo, hi, x_sorted, w3, bias2
    )
    return _sc_row_gather(y_sorted, inv)


# speedup vs baseline: 1.2720x; 1.0139x over previous
import functools

import jax
import jax.numpy as jnp
from jax import lax
from jax.experimental import pallas as pl
from jax.experimental.pallas import tpu as pltpu
from jax.experimental.pallas import tpu_sc as plsc

IN_F = 256
OUT_F = 256
NUM_E = 64
BATCH = 2048

TILE = 128
NUM_TILES = BATCH // TILE
G = NUM_TILES + NUM_E - 1

SC_CORES = 2
SC_SUBCORES = 16
SC_WORKERS = SC_CORES * SC_SUBCORES


def _sc_row_gather(table, idx):
    n = idx.shape[0]
    d = table.shape[1]
    rows_per_w = n // SC_WORKERS
    mesh = plsc.VectorSubcoreMesh(core_axis_name="c", subcore_axis_name="s")

    @functools.partial(
        pl.kernel,
        out_type=jax.ShapeDtypeStruct((n, d), table.dtype),
        mesh=mesh,
        scratch_types=[
            pltpu.VMEM((rows_per_w,), jnp.int32),
            pltpu.VMEM((rows_per_w, d), table.dtype),
            pltpu.SemaphoreType.DMA,
        ],
    )
    def k(table_hbm, idx_hbm, out_hbm, idx_v, rows_v, sem):
        wid = lax.axis_index("s") * SC_CORES + lax.axis_index("c")
        base = wid * rows_per_w
        pltpu.sync_copy(idx_hbm.at[pl.ds(base, rows_per_w)], idx_v)
        pltpu.async_copy(table_hbm.at[idx_v], rows_v, sem).wait()
        pltpu.sync_copy(rows_v, out_hbm.at[pl.ds(base, rows_per_w)])

    return k(table, idx)


NBUF = 8


def _gmm_body(
    n_r, tile_r, e_r, lo_r, hi_r, x_ref, w_hbm, b_ref, o_ref, wbuf, sems
):
    n = n_r[0]

    def start(slot, j):
        pltpu.make_async_copy(
            w_hbm.at[pl.ds(e_r[j], 1)], wbuf.at[pl.ds(slot, 1)], sems.at[slot]
        ).start()

    def wait(slot):
        pltpu.make_async_copy(
            w_hbm.at[pl.ds(0, 1)], wbuf.at[pl.ds(slot, 1)], sems.at[slot]
        ).wait()

    o_ref[...] = jnp.zeros((BATCH, OUT_F), jnp.float32)
    for b in range(NBUF):
        @pl.when(b < n)
        def _(b=b):
            start(b, b)

    def round_body(rnd, _):
        j0 = rnd * NBUF
        for b in range(NBUF):
            @pl.when(j0 + b < n)
            def _(b=b):
                j = j0 + b
                wait(b)
                t = tile_r[j]
                lo = lo_r[j]
                hi = hi_r[j]
                base = t * TILE
                rows = base + lax.broadcasted_iota(jnp.int32, (TILE, 1), 0)
                mask = (rows >= lo) & (rows < hi)
                x = jnp.where(mask, x_ref[pl.ds(base, TILE), :], 0.0).astype(
                    jnp.bfloat16
                )
                w = wbuf[pl.ds(b, 1)][0]
                acc = lax.dot_general(
                    x, w, (((1,), (1,)), ((), ())),
                    preferred_element_type=jnp.float32,
                )
                brow = b_ref[pl.ds(e_r[j], 1), :]
                acc = acc + jnp.where(mask, brow, 0.0)
                o_ref[pl.ds(base, TILE), :] += acc

                @pl.when(j + NBUF < n)
                def _():
                    start(b, j + NBUF)
        return 0

    nrounds = (n + NBUF - 1) // NBUF
    lax.fori_loop(0, nrounds, round_body, 0)


def _grouped_matmul(n_items, tile_of, e_of, lo, hi, x_sorted, w3, bias2):
    grid_spec = pltpu.PrefetchScalarGridSpec(
        num_scalar_prefetch=5,
        grid=(1,),
        in_specs=[
            pl.BlockSpec((BATCH, IN_F), lambda g, *_: (0, 0)),
            pl.BlockSpec(memory_space=pltpu.MemorySpace.HBM),
            pl.BlockSpec((NUM_E, OUT_F), lambda g, *_: (0, 0)),
        ],
        out_specs=pl.BlockSpec((BATCH, OUT_F), lambda g, *_: (0, 0)),
        scratch_shapes=[
            pltpu.VMEM((NBUF, OUT_F, IN_F), jnp.bfloat16),
            pltpu.SemaphoreType.DMA((NBUF,)),
        ],
    )
    return pl.pallas_call(
        _gmm_body,
        grid_spec=grid_spec,
        out_shape=jax.ShapeDtypeStruct((BATCH, OUT_F), jnp.float32),
        compiler_params=pltpu.CompilerParams(dimension_semantics=("arbitrary",)),
    )(n_items, tile_of, e_of, lo, hi, x_sorted, w3, bias2)


def kernel(input, route_index, weight, weight_fact, bias, bias_fact):
    r = route_index.astype(jnp.int32)
    perm = jnp.argsort(r).astype(jnp.int32)
    inv = jnp.zeros((BATCH,), jnp.int32).at[perm].set(
        jnp.arange(BATCH, dtype=jnp.int32)
    )

    counts = jnp.bincount(r, length=NUM_E)
    off = jnp.concatenate(
        [jnp.zeros((1,), jnp.int32), jnp.cumsum(counts).astype(jnp.int32)]
    )
    first_t = off[:NUM_E] // TILE
    nonempty = counts > 0
    last_t = jnp.where(nonempty, (off[1:] - 1) // TILE, 0)
    ntiles = jnp.where(nonempty, last_t - first_t + 1, 0)
    cum = jnp.cumsum(ntiles)
    starts = cum - ntiles
    gids = jnp.arange(G)
    e_g = jnp.searchsorted(cum, gids, side="right")
    valid = e_g < NUM_E
    e_safe = jnp.where(valid, e_g, 0).astype(jnp.int32)
    t_g = first_t[e_safe] + (gids - starts[e_safe])
    tile_of = jnp.where(valid, t_g, NUM_TILES - 1).astype(jnp.int32)
    lo = jnp.where(valid, off[e_safe], 0).astype(jnp.int32)
    hi = jnp.where(valid, off[e_safe + 1], 0).astype(jnp.int32)

    x_sorted = _sc_row_gather(input, perm)
    w3 = (weight + weight_fact).reshape(NUM_E, OUT_F, IN_F).astype(jnp.bfloat16)
    bias2 = bias + bias_fact
    n_items = cum[NUM_E - 1 :].astype(jnp.int32)
    y_sorted = _grouped_matmul(
        n_items, tile_of, e_safe, lo, hi, x_sorted, w3, bias2
    )
    return _sc_row_gather(y_sorted, inv)


# speedup vs baseline: 1.6057x; 1.2623x over previous
import functools

import jax
import jax.numpy as jnp
from jax import lax
from jax.experimental import pallas as pl
from jax.experimental.pallas import tpu as pltpu
from jax.experimental.pallas import tpu_sc as plsc

IN_F = 256
OUT_F = 256
NUM_E = 64
BATCH = 2048

TILE = 128
NUM_TILES = BATCH // TILE
G = NUM_TILES + NUM_E - 1

SC_CORES = 2
SC_SUBCORES = 16
SC_WORKERS = SC_CORES * SC_SUBCORES


def _sc_row_gather(table, idx):
    n = idx.shape[0]
    d = table.shape[1]
    rows_per_w = n // SC_WORKERS
    mesh = plsc.VectorSubcoreMesh(core_axis_name="c", subcore_axis_name="s")

    @functools.partial(
        pl.kernel,
        out_type=jax.ShapeDtypeStruct((n, d), table.dtype),
        mesh=mesh,
        scratch_types=[
            pltpu.VMEM((rows_per_w,), jnp.int32),
            pltpu.VMEM((rows_per_w, d), table.dtype),
            pltpu.SemaphoreType.DMA,
        ],
    )
    def k(table_hbm, idx_hbm, out_hbm, idx_v, rows_v, sem):
        wid = lax.axis_index("s") * SC_CORES + lax.axis_index("c")
        base = wid * rows_per_w
        pltpu.sync_copy(idx_hbm.at[pl.ds(base, rows_per_w)], idx_v)
        pltpu.async_copy(table_hbm.at[idx_v], rows_v, sem).wait()
        pltpu.sync_copy(rows_v, out_hbm.at[pl.ds(base, rows_per_w)])

    return k(table, idx)


def _sc_row_scatter(rows, idx):
    n = idx.shape[0]
    d = rows.shape[1]
    rows_per_w = n // SC_WORKERS
    mesh = plsc.VectorSubcoreMesh(core_axis_name="c", subcore_axis_name="s")

    @functools.partial(
        pl.kernel,
        out_type=jax.ShapeDtypeStruct((n, d), rows.dtype),
        mesh=mesh,
        scratch_types=[
            pltpu.VMEM((rows_per_w,), jnp.int32),
            pltpu.VMEM((rows_per_w, d), rows.dtype),
            pltpu.SemaphoreType.DMA,
        ],
    )
    def k(rows_hbm, idx_hbm, out_hbm, idx_v, rows_v, sem):
        wid = lax.axis_index("s") * SC_CORES + lax.axis_index("c")
        base = wid * rows_per_w
        pltpu.sync_copy(idx_hbm.at[pl.ds(base, rows_per_w)], idx_v)
        pltpu.sync_copy(rows_hbm.at[pl.ds(base, rows_per_w)], rows_v)
        pltpu.async_copy(rows_v, out_hbm.at[idx_v], sem).wait()

    return k(rows, idx)


NBUF = 8


def _gmm_body(
    n_r, tile_r, e_r, lo_r, hi_r, x_ref, w_hbm, b_ref, o_ref, wbuf, sems
):
    n = n_r[0]

    def start(slot, j):
        pltpu.make_async_copy(
            w_hbm.at[pl.ds(e_r[j], 1)], wbuf.at[pl.ds(slot, 1)], sems.at[slot]
        ).start()

    def wait(slot):
        pltpu.make_async_copy(
            w_hbm.at[pl.ds(0, 1)], wbuf.at[pl.ds(slot, 1)], sems.at[slot]
        ).wait()

    o_ref[...] = jnp.zeros((BATCH, OUT_F), jnp.float32)
    for b in range(NBUF):
        @pl.when(b < n)
        def _(b=b):
            start(b, b)

    def round_body(rnd, _):
        j0 = rnd * NBUF
        for b in range(NBUF):
            @pl.when(j0 + b < n)
            def _(b=b):
                j = j0 + b
                wait(b)
                t = tile_r[j]
                lo = lo_r[j]
                hi = hi_r[j]
                base = t * TILE
                rows = base + lax.broadcasted_iota(jnp.int32, (TILE, 1), 0)
                mask = (rows >= lo) & (rows < hi)
                x = jnp.where(mask, x_ref[pl.ds(base, TILE), :], 0.0).astype(
                    jnp.bfloat16
                )
                w = wbuf[pl.ds(b, 1)][0]
                acc = lax.dot_general(
                    x, w, (((1,), (1,)), ((), ())),
                    preferred_element_type=jnp.float32,
                )
                brow = b_ref[pl.ds(e_r[j], 1), :]
                acc = acc + jnp.where(mask, brow, 0.0)
                o_ref[pl.ds(base, TILE), :] += acc

                @pl.when(j + NBUF < n)
                def _():
                    start(b, j + NBUF)
        return 0

    nrounds = (n + NBUF - 1) // NBUF
    lax.fori_loop(0, nrounds, round_body, 0)


def _grouped_matmul(n_items, tile_of, e_of, lo, hi, x_sorted, w3, bias2):
    grid_spec = pltpu.PrefetchScalarGridSpec(
        num_scalar_prefetch=5,
        grid=(1,),
        in_specs=[
            pl.BlockSpec((BATCH, IN_F), lambda g, *_: (0, 0)),
            pl.BlockSpec(memory_space=pltpu.MemorySpace.HBM),
            pl.BlockSpec((NUM_E, OUT_F), lambda g, *_: (0, 0)),
        ],
        out_specs=pl.BlockSpec((BATCH, OUT_F), lambda g, *_: (0, 0)),
        scratch_shapes=[
            pltpu.VMEM((NBUF, OUT_F, IN_F), jnp.bfloat16),
            pltpu.SemaphoreType.DMA((NBUF,)),
        ],
    )
    return pl.pallas_call(
        _gmm_body,
        grid_spec=grid_spec,
        out_shape=jax.ShapeDtypeStruct((BATCH, OUT_F), jnp.float32),
        compiler_params=pltpu.CompilerParams(dimension_semantics=("arbitrary",)),
    )(n_items, tile_of, e_of, lo, hi, x_sorted, w3, bias2)


def kernel(input, route_index, weight, weight_fact, bias, bias_fact):
    r = route_index.astype(jnp.int32)
    eids = jnp.arange(NUM_E, dtype=jnp.int32)

    H = r[:, None] == eids[None, :]
    tri = jnp.tril(jnp.ones((BATCH, BATCH), jnp.bfloat16))
    C = lax.dot_general(
        tri,
        H.astype(jnp.bfloat16),
        (((1,), (0,)), ((), ())),
        preferred_element_type=jnp.float32,
    )
    counts = C[BATCH - 1].astype(jnp.int32)
    off_incl = jnp.cumsum(counts)
    off_excl = off_incl - counts
    rank = jnp.sum(
        jnp.where(H, off_excl[None, :].astype(jnp.float32) + C - 1.0, 0.0),
        axis=1,
    ).astype(jnp.int32)

    nonempty = counts > 0
    first_t = off_excl // TILE
    last_t = jnp.where(nonempty, (off_incl - 1) // TILE, 0)
    ntiles = jnp.where(nonempty, last_t - first_t + 1, 0)
    cum = jnp.cumsum(ntiles)
    starts = cum - ntiles
    gids = jnp.arange(G, dtype=jnp.int32)
    e_g = jnp.sum((cum[None, :] <= gids[:, None]).astype(jnp.int32), axis=1)
    valid = e_g < NUM_E
    onehot = e_g[:, None] == eids[None, :]

    def pick(v):
        return jnp.sum(jnp.where(onehot, v[None, :], 0), axis=1)

    t_g = pick(first_t) + (gids - pick(starts))
    tile_of = jnp.where(valid, t_g, NUM_TILES - 1).astype(jnp.int32)
    e_safe = jnp.where(valid, e_g, 0).astype(jnp.int32)
    lo = jnp.where(valid, pick(off_excl), 0).astype(jnp.int32)
    hi = jnp.where(valid, pick(off_incl), 0).astype(jnp.int32)
    n_items = cum[NUM_E - 1 :].astype(jnp.int32)

    x_sorted = _sc_row_scatter(input, rank)
    w3 = (weight + weight_fact).reshape(NUM_E, OUT_F, IN_F).astype(jnp.bfloat16)
    bias2 = bias + bias_fact
    y_sorted = _grouped_matmul(
        n_items, tile_of, e_safe, lo, hi, x_sorted, w3, bias2
    )
    return _sc_row_gather(y_sorted, rank)


# speedup vs baseline: 1.6083x; 1.0017x over previous
import functools

import jax
import jax.numpy as jnp
from jax import lax
from jax.experimental import pallas as pl
from jax.experimental.pallas import tpu as pltpu
from jax.experimental.pallas import tpu_sc as plsc

IN_F = 256
OUT_F = 256
NUM_E = 64
BATCH = 2048

TILE = 128
NUM_TILES = BATCH // TILE
G = NUM_TILES + NUM_E - 1

SC_CORES = 2
SC_SUBCORES = 16
SC_WORKERS = SC_CORES * SC_SUBCORES


def _sc_row_gather(table, idx):
    n = idx.shape[0]
    d = table.shape[1]
    rows_per_w = n // SC_WORKERS
    mesh = plsc.VectorSubcoreMesh(core_axis_name="c", subcore_axis_name="s")

    @functools.partial(
        pl.kernel,
        out_type=jax.ShapeDtypeStruct((n, d), table.dtype),
        mesh=mesh,
        scratch_types=[
            pltpu.VMEM((rows_per_w,), jnp.int32),
            pltpu.VMEM((rows_per_w, d), table.dtype),
            pltpu.SemaphoreType.DMA,
        ],
    )
    def k(table_hbm, idx_hbm, out_hbm, idx_v, rows_v, sem):
        wid = lax.axis_index("s") * SC_CORES + lax.axis_index("c")
        base = wid * rows_per_w
        pltpu.sync_copy(idx_hbm.at[pl.ds(base, rows_per_w)], idx_v)
        pltpu.async_copy(table_hbm.at[idx_v], rows_v, sem).wait()
        pltpu.sync_copy(rows_v, out_hbm.at[pl.ds(base, rows_per_w)])

    return k(table, idx)


def _sc_row_scatter(rows, idx):
    n = idx.shape[0]
    d = rows.shape[1]
    rows_per_w = n // SC_WORKERS
    mesh = plsc.VectorSubcoreMesh(core_axis_name="c", subcore_axis_name="s")

    @functools.partial(
        pl.kernel,
        out_type=jax.ShapeDtypeStruct((n, d), rows.dtype),
        mesh=mesh,
        scratch_types=[
            pltpu.VMEM((rows_per_w,), jnp.int32),
            pltpu.VMEM((rows_per_w, d), rows.dtype),
            pltpu.SemaphoreType.DMA,
        ],
    )
    def k(rows_hbm, idx_hbm, out_hbm, idx_v, rows_v, sem):
        wid = lax.axis_index("s") * SC_CORES + lax.axis_index("c")
        base = wid * rows_per_w
        pltpu.sync_copy(idx_hbm.at[pl.ds(base, rows_per_w)], idx_v)
        pltpu.sync_copy(rows_hbm.at[pl.ds(base, rows_per_w)], rows_v)
        pltpu.async_copy(rows_v, out_hbm.at[idx_v], sem).wait()

    return k(rows, idx)


NBUF = 8


HBATCH = BATCH // 2


def _gmm_body(
    se_r, tile_r, e_r, lo_r, hi_r, x_ref, w_hbm, b_ref, o_ref, wbuf, sems
):
    c = pl.program_id(0)
    j_lo = se_r[c]
    j_hi = se_r[c + 1]

    def start(slot, j):
        pltpu.make_async_copy(
            w_hbm.at[pl.ds(e_r[j], 1)], wbuf.at[pl.ds(slot, 1)], sems.at[slot]
        ).start()

    def wait(slot):
        pltpu.make_async_copy(
            w_hbm.at[pl.ds(0, 1)], wbuf.at[pl.ds(slot, 1)], sems.at[slot]
        ).wait()

    o_ref[...] = jnp.zeros((HBATCH, OUT_F), jnp.float32)
    for b in range(NBUF):
        @pl.when(j_lo + b < j_hi)
        def _(b=b):
            start(b, j_lo + b)

    def round_body(rnd, _):
        j0 = j_lo + rnd * NBUF
        for b in range(NBUF):
            @pl.when(j0 + b < j_hi)
            def _(b=b):
                j = j0 + b
                wait(b)
                t = tile_r[j]
                lo = lo_r[j]
                hi = hi_r[j]
                gbase = t * TILE
                base = gbase - c * HBATCH
                rows = gbase + lax.broadcasted_iota(jnp.int32, (TILE, 1), 0)
                mask = (rows >= lo) & (rows < hi)
                x = jnp.where(mask, x_ref[pl.ds(base, TILE), :], 0.0).astype(
                    jnp.bfloat16
                )
                w = wbuf[pl.ds(b, 1)][0]
                acc = lax.dot_general(
                    x, w, (((1,), (1,)), ((), ())),
                    preferred_element_type=jnp.float32,
                )
                brow = b_ref[pl.ds(e_r[j], 1), :]
                acc = acc + jnp.where(mask, brow, 0.0)
                o_ref[pl.ds(base, TILE), :] += acc

                @pl.when(j + NBUF < j_hi)
                def _():
                    start(b, j + NBUF)
        return 0

    nrounds = (j_hi - j_lo + NBUF - 1) // NBUF
    lax.fori_loop(0, nrounds, round_body, 0)


def _grouped_matmul(split, tile_of, e_of, lo, hi, x_sorted, w3, bias2):
    grid_spec = pltpu.PrefetchScalarGridSpec(
        num_scalar_prefetch=5,
        grid=(2,),
        in_specs=[
            pl.BlockSpec((HBATCH, IN_F), lambda c, *_: (c, 0)),
            pl.BlockSpec(memory_space=pltpu.MemorySpace.HBM),
            pl.BlockSpec((NUM_E, OUT_F), lambda c, *_: (0, 0)),
        ],
        out_specs=pl.BlockSpec((HBATCH, OUT_F), lambda c, *_: (c, 0)),
        scratch_shapes=[
            pltpu.VMEM((NBUF, OUT_F, IN_F), jnp.bfloat16),
            pltpu.SemaphoreType.DMA((NBUF,)),
        ],
    )
    return pl.pallas_call(
        _gmm_body,
        grid_spec=grid_spec,
        out_shape=jax.ShapeDtypeStruct((BATCH, OUT_F), jnp.float32),
        compiler_params=pltpu.CompilerParams(dimension_semantics=("parallel",)),
    )(split, tile_of, e_of, lo, hi, x_sorted, w3, bias2)


def kernel(input, route_index, weight, weight_fact, bias, bias_fact):
    r = route_index.astype(jnp.int32)
    eids = jnp.arange(NUM_E, dtype=jnp.int32)

    H = r[:, None] == eids[None, :]
    tri = jnp.tril(jnp.ones((BATCH, BATCH), jnp.bfloat16))
    C = lax.dot_general(
        tri,
        H.astype(jnp.bfloat16),
        (((1,), (0,)), ((), ())),
        preferred_element_type=jnp.float32,
    )
    counts = C[BATCH - 1].astype(jnp.int32)
    off_incl = jnp.cumsum(counts)
    off_excl = off_incl - counts
    rank = jnp.sum(
        jnp.where(H, off_excl[None, :].astype(jnp.float32) + C - 1.0, 0.0),
        axis=1,
    ).astype(jnp.int32)

    nonempty = counts > 0
    first_t = off_excl // TILE
    last_t = jnp.where(nonempty, (off_incl - 1) // TILE, 0)
    ntiles = jnp.where(nonempty, last_t - first_t + 1, 0)
    cum = jnp.cumsum(ntiles)
    starts = cum - ntiles
    gids = jnp.arange(G, dtype=jnp.int32)
    e_g = jnp.sum((cum[None, :] <= gids[:, None]).astype(jnp.int32), axis=1)
    valid = e_g < NUM_E
    onehot = e_g[:, None] == eids[None, :]

    def pick(v):
        return jnp.sum(jnp.where(onehot, v[None, :], 0), axis=1)

    t_g = pick(first_t) + (gids - pick(starts))
    tile_of = jnp.where(valid, t_g, NUM_TILES - 1).astype(jnp.int32)
    e_safe = jnp.where(valid, e_g, 0).astype(jnp.int32)
    lo = jnp.where(valid, pick(off_excl), 0).astype(jnp.int32)
    hi = jnp.where(valid, pick(off_incl), 0).astype(jnp.int32)
    n_items = cum[NUM_E - 1]
    s = jnp.sum((valid & (tile_of < NUM_TILES // 2)).astype(jnp.int32))
    split = jnp.stack([jnp.int32(0), s, n_items]).astype(jnp.int32)

    x_sorted = _sc_row_scatter(input, rank)
    w3 = (weight + weight_fact).reshape(NUM_E, OUT_F, IN_F).astype(jnp.bfloat16)
    bias2 = bias + bias_fact
    y_sorted = _grouped_matmul(
        split, tile_of, e_safe, lo, hi, x_sorted, w3, bias2
    )
    return _sc_row_gather(y_sorted, rank)
